# baseline (speedup 1.0000x reference)
import functools

import jax
import jax.numpy as jnp
from jax import lax
from jax.experimental import pallas as pl
from jax.experimental.pallas import tpu as pltpu
from jax.experimental.pallas import tpu_sc as plsc

_S = 64.0
_M = 0.35
_LANES = 128


@functools.lru_cache(maxsize=None)
def _make_target_gather(B, C):
    info = plsc.get_sparse_core_info()
    NC, NS, L = info.num_cores, info.num_subcores, info.num_lanes
    NW = NC * NS
    assert B % NW == 0 and (B // NW) % L == 0
    bpw = B // NW
    mesh = plsc.VectorSubcoreMesh(core_axis_name="c", subcore_axis_name="s")

    @functools.partial(
        pl.kernel,
        mesh=mesh,
        out_type=jax.ShapeDtypeStruct((B, 8, _LANES), jnp.float32),
        scratch_types=[
            pltpu.VMEM((bpw,), jnp.int32),
            pltpu.SemaphoreType.DMA,
        ],
    )
    def gather_k(x_hbm, off_hbm, out_hbm, offv, sem):
        wid = lax.axis_index("s") * NC + lax.axis_index("c")
        base = wid * bpw
        pltpu.sync_copy(off_hbm.at[pl.ds(base, bpw)], offv)
        cps = []
        for c in range(bpw // L):
            chunk = offv[pl.ds(c * L, L)]
            for k in range(L):
                r = c * L + k
                o = pl.multiple_of(chunk[k], _LANES)
                cps.append(
                    pltpu.async_copy(
                        x_hbm.at[pl.ds(base + 8 * (r // 8), 8),
                                 pl.ds(o, _LANES)],
                        out_hbm.at[base + r], sem))
        for cp in cps:
            cp.wait()

    return gather_k


@functools.lru_cache(maxsize=None)
def _make_stream(B, C, R, W):
    Cout = C - 1
    ncols = pl.cdiv(Cout, W)
    WB = W // _LANES
    nb_max = (C - _LANES) // _LANES

    def body(x_ref, xn_ref, gran_ref, lane_ref, lab_ref, diff_ref, st_ref,
             stm_ref, sm_ref, t_scr):
        j = pl.program_id(1)

        @pl.when(j == 0)
        def _pick_target():
            g = gran_ref[...]
            i0 = lax.broadcasted_iota(jnp.int32, (R, 8, _LANES), 0)
            i1 = lax.broadcasted_iota(jnp.int32, (R, 8, _LANES), 1)
            i2 = lax.broadcasted_iota(jnp.int32, (R, 8, _LANES), 2)
            lane3 = lane_ref[...].reshape(R, 1, 1)
            m = (i1 == (i0 % 8)) & (i2 == lane3)
            t_scr[...] = jnp.sum(jnp.where(m, g, 0.0),
                                 axis=(1, 2)).reshape(R, 1)

        x = x_ref[...]
        xn = xn_ref[:, :1]
        lab = lab_ref[...]
        t = t_scr[...]
        ft = t - _M
        shifted = jnp.concatenate([x[:, 1:], xn], axis=1)
        col = lax.broadcasted_iota(jnp.int32, (R, W), 1) + j * W
        sel = jnp.where(col < lab, x, shifted)
        diff_ref[...] = _S * (sel - ft)
        st = jnp.sqrt(jnp.maximum(1.0 - t * t, 0.0))
        stm = jnp.sqrt(jnp.maximum(1.0 - ft * ft, 0.0))
        st_ref[...] = st
        stm_ref[...] = stm
        sm_ref[...] = stm * t - ft * st

    return pl.pallas_call(
        body,
        grid=(B // R, ncols),
        in_specs=[
            pl.BlockSpec((R, W), lambda i, j: (i, j)),
            pl.BlockSpec((R, _LANES),
                         lambda i, j: (i, jnp.minimum((j + 1) * WB, nb_max))),
            pl.BlockSpec((R, 8, _LANES), lambda i, j: (i, 0, 0)),
            pl.BlockSpec((R, 1), lambda i, j: (i, 0)),
            pl.BlockSpec((R, 1), lambda i, j: (i, 0)),
        ],
        out_specs=[
            pl.BlockSpec((R, W), lambda i, j: (i, j)),
            pl.BlockSpec((R, 1), lambda i, j: (i, 0)),
            pl.BlockSpec((R, 1), lambda i, j: (i, 0)),
            pl.BlockSpec((R, 1), lambda i, j: (i, 0)),
        ],
        out_shape=[
            jax.ShapeDtypeStruct((B, Cout), jnp.float32),
            jax.ShapeDtypeStruct((B, 1), jnp.float32),
            jax.ShapeDtypeStruct((B, 1), jnp.float32),
            jax.ShapeDtypeStruct((B, 1), jnp.float32),
        ],
        scratch_shapes=[pltpu.VMEM((R, 1), jnp.float32)],
        compiler_params=pltpu.CompilerParams(
            dimension_semantics=("parallel", "arbitrary"),
        ),
    )


def kernel(logits, labels):
    B, C = logits.shape
    labels = labels.astype(jnp.int32)
    off = (labels // _LANES) * _LANES
    lane = labels - off
    gran = _make_target_gather(B, C)(logits, off)
    diff, st, stm, sm = _make_stream(B, C, 512, 4096)(
        logits, logits, gran, lane.reshape(B, 1), labels.reshape(B, 1)
    )
    return diff, st.reshape(B), stm.reshape(B), sm.reshape(B)


# speedup vs baseline: 1.1232x; 1.1232x over previous
import functools

import jax
import jax.numpy as jnp
from jax import lax
from jax.experimental import pallas as pl
from jax.experimental.pallas import tpu as pltpu
from jax.experimental.pallas import tpu_sc as plsc

_S = 64.0
_M = 0.35
_LANES = 128


@functools.lru_cache(maxsize=None)
def _make_target_gather(B, C):
    info = plsc.get_sparse_core_info()
    NC, NS, L = info.num_cores, info.num_subcores, info.num_lanes
    NW = NC * NS
    assert B % NW == 0 and (B // NW) % L == 0
    bpw = B // NW
    mesh = plsc.VectorSubcoreMesh(core_axis_name="c", subcore_axis_name="s")

    @functools.partial(
        pl.kernel,
        mesh=mesh,
        out_type=jax.ShapeDtypeStruct((B, 8, _LANES), jnp.float32),
        scratch_types=[
            pltpu.VMEM((bpw,), jnp.int32),
            pltpu.VMEM((bpw, 8, _LANES), jnp.float32),
            pltpu.SemaphoreType.DMA,
        ],
    )
    def gather_k(x_hbm, off_hbm, out_hbm, offv, slabv, sem):
        wid = lax.axis_index("s") * NC + lax.axis_index("c")
        base = wid * bpw
        pltpu.sync_copy(off_hbm.at[pl.ds(base, bpw)], offv)
        cps = []
        for c in range(bpw // L):
            chunk = offv[pl.ds(c * L, L)]
            for k in range(L):
                r = c * L + k
                o = pl.multiple_of(chunk[k], _LANES)
                cps.append(
                    pltpu.async_copy(
                        x_hbm.at[pl.ds(base + 8 * (r // 8), 8),
                                 pl.ds(o, _LANES)],
                        slabv.at[r], sem))
        for cp in cps:
            cp.wait()
        pltpu.sync_copy(slabv, out_hbm.at[pl.ds(base, bpw)])

    return gather_k


@functools.lru_cache(maxsize=None)
def _make_stream(B, C, R, W):
    Cout = C - 1
    ncols = pl.cdiv(Cout, W)
    WB = W // _LANES
    nb_max = (C - _LANES) // _LANES

    def body(x_ref, xn_ref, gran_ref, lane_ref, lab_ref, diff_ref, st_ref,
             stm_ref, sm_ref, t_scr):
        j = pl.program_id(1)

        @pl.when(j == 0)
        def _pick_target():
            g = gran_ref[...]
            i0 = lax.broadcasted_iota(jnp.int32, (R, 8, _LANES), 0)
            i1 = lax.broadcasted_iota(jnp.int32, (R, 8, _LANES), 1)
            i2 = lax.broadcasted_iota(jnp.int32, (R, 8, _LANES), 2)
            lane3 = lane_ref[...].reshape(R, 1, 1)
            m = (i1 == (i0 % 8)) & (i2 == lane3)
            t_scr[...] = jnp.sum(jnp.where(m, g, 0.0),
                                 axis=(1, 2)).reshape(R, 1)

        x = x_ref[...]
        xn = xn_ref[:, :1]
        lab = lab_ref[...]
        t = t_scr[...]
        ft = t - _M
        shifted = jnp.concatenate([x[:, 1:], xn], axis=1)
        col = lax.broadcasted_iota(jnp.int32, (R, W), 1) + j * W
        sel = jnp.where(col < lab, x, shifted)
        diff_ref[...] = _S * (sel - ft)
        st = jnp.sqrt(jnp.maximum(1.0 - t * t, 0.0))
        stm = jnp.sqrt(jnp.maximum(1.0 - ft * ft, 0.0))
        st_ref[...] = st
        stm_ref[...] = stm
        sm_ref[...] = stm * t - ft * st

    return pl.pallas_call(
        body,
        grid=(B // R, ncols),
        in_specs=[
            pl.BlockSpec((R, W), lambda i, j: (i, j)),
            pl.BlockSpec((R, _LANES),
                         lambda i, j: (i, jnp.minimum((j + 1) * WB, nb_max))),
            pl.BlockSpec((R, 8, _LANES), lambda i, j: (i, 0, 0)),
            pl.BlockSpec((R, 1), lambda i, j: (i, 0)),
            pl.BlockSpec((R, 1), lambda i, j: (i, 0)),
        ],
        out_specs=[
            pl.BlockSpec((R, W), lambda i, j: (i, j)),
            pl.BlockSpec((R, 1), lambda i, j: (i, 0)),
            pl.BlockSpec((R, 1), lambda i, j: (i, 0)),
            pl.BlockSpec((R, 1), lambda i, j: (i, 0)),
        ],
        out_shape=[
            jax.ShapeDtypeStruct((B, Cout), jnp.float32),
            jax.ShapeDtypeStruct((B, 1), jnp.float32),
            jax.ShapeDtypeStruct((B, 1), jnp.float32),
            jax.ShapeDtypeStruct((B, 1), jnp.float32),
        ],
        scratch_shapes=[pltpu.VMEM((R, 1), jnp.float32)],
        compiler_params=pltpu.CompilerParams(
            dimension_semantics=("parallel", "arbitrary"),
        ),
    )


def kernel(logits, labels):
    B, C = logits.shape
    labels = labels.astype(jnp.int32)
    off = (labels // _LANES) * _LANES
    lane = labels - off
    gran = _make_target_gather(B, C)(logits, off)
    diff, st, stm, sm = _make_stream(B, C, 512, 4096)(
        logits, logits, gran, lane.reshape(B, 1), labels.reshape(B, 1)
    )
    return diff, st.reshape(B), stm.reshape(B), sm.reshape(B)
